# Initial kernel scaffold; baseline (speedup 1.0000x reference)
import jax
import jax.numpy as jnp
from jax.experimental import pallas as pl


def kernel(concatenated_node_features, edge_index, interaction_feature, object_pairs, W_g1, b_g1, W_g2, b_g2, W_cr1, b_cr1, W_cr2, b_cr2, W_lr1, b_lr1, W_lr2, b_lr2, W_mr1, b_mr1, W_mr2, b_mr2):
    raise NotImplementedError("write your pallas kernel here")


# speedup vs baseline: 6.3774x; 6.3774x over previous
import functools

import jax
import jax.numpy as jnp
from jax import lax
from jax.experimental import pallas as pl
from jax.experimental.pallas import tpu as pltpu
from jax.experimental.pallas import tpu_sc as plsc

_B, _N, _E, _D, _EF, _P = 64, 128, 4096, 128, 16, 512
_NC, _NS = 2, 16
_NW = _NC * _NS
_PAIRS = _B * _P
_PW = _PAIRS // _NW
_RPD = 128
_NDMA = _PW // _RPD


def _edge_feature_gather(table, p0, p1):
    mesh = plsc.VectorSubcoreMesh(core_axis_name="c", subcore_axis_name="s")

    @functools.partial(
        pl.kernel,
        out_type=jax.ShapeDtypeStruct((_PAIRS, _EF), jnp.float32),
        mesh=mesh,
        scratch_types=[
            pltpu.VMEM((_NDMA, _RPD), jnp.int32),
            pltpu.VMEM((_NDMA, _RPD), jnp.int32),
            pltpu.VMEM((_NDMA, _RPD), jnp.int32),
            pltpu.VMEM((_PW, _EF), jnp.float32),
            pltpu.SemaphoreType.DMA,
        ],
        compiler_params=pltpu.CompilerParams(use_tc_tiling_on_sc=False),
    )
    def gather_kernel(table_hbm, p0_hbm, p1_hbm, out_hbm,
                      p0_v, p1_v, idx_v, rows_v, sem):
        wid = lax.axis_index("s") * _NC + lax.axis_index("c")
        rbase = wid * _NDMA
        pltpu.sync_copy(p0_hbm.at[pl.ds(rbase, _NDMA)], p0_v)
        pltpu.sync_copy(p1_hbm.at[pl.ds(rbase, _NDMA)], p1_v)
        base = wid * _PW
        for k in range(_NDMA):
            for j in range(_RPD // 16):
                boff = (wid * (_PW // _P) + (k * _RPD) // _P) * (_N * _N)
                i0 = p0_v[k, pl.ds(j * 16, 16)]
                i1 = p1_v[k, pl.ds(j * 16, 16)]
                idx_v[k, pl.ds(j * 16, 16)] = (
                    jnp.full((16,), boff, jnp.int32) + i0 * _N + i1)
        copies = [
            pltpu.async_copy(table_hbm.at[idx_v.at[k]],
                             rows_v.at[pl.ds(k * _RPD, _RPD)], sem)
            for k in range(_NDMA)
        ]
        for c in copies:
            c.wait()
        pltpu.sync_copy(rows_v, out_hbm.at[pl.ds(base, _PW)])

    return gather_kernel(table, p0, p1)


def _tc_body(x_ref, src_ref, dst_ref, p0_ref, p1_ref, ee_ref,
             wg1, bg1, wg2, bg2,
             wl1a, wl1b, bl1, wl2, bl2,
             wc1a, wc1b, bc1, wc2, bc2,
             wm1a, wm1b, bm1, wm2, bm2,
             lr_ref, cr_ref, mr_ref):
    src = src_ref[0]
    dst = dst_ref[0]
    oh_s = (src == lax.broadcasted_iota(jnp.int32, (_E, _N), 1)
            ).astype(jnp.bfloat16)
    oh_dT = (dst == lax.broadcasted_iota(jnp.int32, (_N, _E), 0)
             ).astype(jnp.bfloat16)
    a = jnp.dot(oh_dT, oh_s, preferred_element_type=jnp.float32)
    deg = jnp.maximum(jnp.sum(a, axis=1, keepdims=True), 1.0)

    h = x_ref[0]
    m = jnp.dot(a, h, preferred_element_type=jnp.float32) / deg
    h = jnp.maximum(jnp.dot(m, wg1[...], preferred_element_type=jnp.float32)
                    + bg1[...], 0.0)
    m = jnp.dot(a, h, preferred_element_type=jnp.float32) / deg
    h = jnp.maximum(jnp.dot(m, wg2[...], preferred_element_type=jnp.float32)
                    + bg2[...], 0.0)

    p0 = p0_ref[0]
    p1 = p1_ref[0]
    lane_n = lax.broadcasted_iota(jnp.int32, (_P, _N), 1)
    ohp = (p0 == lane_n).astype(jnp.float32) + (p1 == lane_n).astype(jnp.float32)
    emb = jnp.dot(ohp, h, preferred_element_type=jnp.float32)
    ee = ee_ref[0]

    def head(wa, wb, b1, w2, b2):
        hh = jnp.maximum(
            jnp.dot(emb, wa[...], preferred_element_type=jnp.float32)
            + jnp.dot(ee, wb[...], preferred_element_type=jnp.float32)
            + b1[...], 0.0)
        return jnp.dot(hh, w2[...], preferred_element_type=jnp.float32) + b2[...]

    lr_ref[0] = head(wl1a, wl1b, bl1, wl2, bl2)
    cr_ref[0] = head(wc1a, wc1b, bc1, wc2, bc2)
    mr_ref[0] = head(wm1a, wm1b, bm1, wm2, bm2)


def _full(shape):
    return pl.BlockSpec(shape, lambda b: (0,) * len(shape))


def kernel(concatenated_node_features, edge_index, interaction_feature,
           object_pairs,
           W_g1, b_g1, W_g2, b_g2,
           W_cr1, b_cr1, W_cr2, b_cr2,
           W_lr1, b_lr1, W_lr2, b_lr2,
           W_mr1, b_mr1, W_mr2, b_mr2):
    x = concatenated_node_features
    srcc = edge_index[:, 0, :].reshape(_B, _E, 1)
    dstr = edge_index[:, 1, :].reshape(_B, 1, _E)
    p0 = object_pairs[:, :, 0]
    p1 = object_pairs[:, :, 1]

    table = interaction_feature.reshape(_B * _N * _N, _EF)
    ee = _edge_feature_gather(table,
                              p0.reshape(_PAIRS // _RPD, _RPD),
                              p1.reshape(_PAIRS // _RPD, _RPD))
    ee = ee.reshape(_B, _P, _EF)

    heads = []
    for w1, b1, w2, b2 in ((W_lr1, b_lr1, W_lr2, b_lr2),
                           (W_cr1, b_cr1, W_cr2, b_cr2),
                           (W_mr1, b_mr1, W_mr2, b_mr2)):
        heads += [w1[:_D], w1[_D:], b1.reshape(1, -1), w2, b2.reshape(1, -1)]

    out_shapes = [jax.ShapeDtypeStruct((_B, _P, c), jnp.float32)
                  for c in (10, 26, 5)]
    per_b = lambda shape: pl.BlockSpec(shape, lambda b: (b, 0, 0))

    lr, cr, mr = pl.pallas_call(
        _tc_body,
        grid=(_B,),
        in_specs=[
            per_b((1, _N, _D)),
            per_b((1, _E, 1)),
            per_b((1, 1, _E)),
            per_b((1, _P, 1)),
            per_b((1, _P, 1)),
            per_b((1, _P, _EF)),
            _full((_D, _D)), _full((1, _D)), _full((_D, _D)), _full((1, _D)),
            _full((_D, 64)), _full((_EF, 64)), _full((1, 64)),
            _full((64, 10)), _full((1, 10)),
            _full((_D, 64)), _full((_EF, 64)), _full((1, 64)),
            _full((64, 26)), _full((1, 26)),
            _full((_D, 64)), _full((_EF, 64)), _full((1, 64)),
            _full((64, 5)), _full((1, 5)),
        ],
        out_specs=[per_b((1, _P, 10)), per_b((1, _P, 26)), per_b((1, _P, 5))],
        out_shape=out_shapes,
    )(x, srcc, dstr, object_pairs[:, :, 0].reshape(_B, _P, 1),
      object_pairs[:, :, 1].reshape(_B, _P, 1), ee,
      W_g1, b_g1.reshape(1, -1), W_g2, b_g2.reshape(1, -1), *heads)
    return (lr, cr, mr)


# speedup vs baseline: 7.3185x; 1.1476x over previous
import functools

import jax
import jax.numpy as jnp
from jax import lax
from jax.experimental import pallas as pl
from jax.experimental.pallas import tpu as pltpu
from jax.experimental.pallas import tpu_sc as plsc

_B, _N, _E, _D, _EF, _P = 64, 128, 4096, 128, 16, 512
_NC, _NS = 2, 16
_NW = _NC * _NS
_PAIRS = _B * _P
_PW = _PAIRS // _NW
_RPD = 128
_NDMA = _PW // _RPD


def _edge_feature_gather(table, pairs_flat):
    mesh = plsc.VectorSubcoreMesh(core_axis_name="c", subcore_axis_name="s")

    @functools.partial(
        pl.kernel,
        out_type=jax.ShapeDtypeStruct((_PAIRS, _EF), jnp.float32),
        mesh=mesh,
        scratch_types=[
            pltpu.VMEM((2 * _PW,), jnp.int32),
            pltpu.VMEM((_NDMA, _RPD), jnp.int32),
            pltpu.VMEM((_PW, _EF), jnp.float32),
            pltpu.SemaphoreType.DMA,
        ],
        compiler_params=pltpu.CompilerParams(use_tc_tiling_on_sc=False,
                                             needs_layout_passes=False),
    )
    def gather_kernel(table_hbm, pairs_hbm, out_hbm, pq_v, idx_v, rows_v, sem):
        wid = lax.axis_index("s") * _NC + lax.axis_index("c")
        base = wid * _PW
        pltpu.sync_copy(pairs_hbm.at[pl.ds(2 * base, 2 * _PW)], pq_v)
        lane = lax.iota(jnp.int32, 16)
        for c in range(_PW // 16):
            gidx = 2 * (c * 16) + 2 * lane
            i0 = plsc.load_gather(pq_v, [gidx])
            i1 = plsc.load_gather(pq_v, [gidx + 1])
            boff = (wid * (_PW // _P) + (c * 16) // _P) * (_N * _N)
            idx_v[c // 8, pl.ds((c % 8) * 16, 16)] = (
                jnp.full((16,), boff, jnp.int32) + i0 * _N + i1)
        copies = [
            pltpu.async_copy(table_hbm.at[idx_v.at[k]],
                             rows_v.at[pl.ds(k * _RPD, _RPD)], sem)
            for k in range(_NDMA)
        ]
        for c in copies:
            c.wait()
        pltpu.sync_copy(rows_v, out_hbm.at[pl.ds(base, _PW)])

    return gather_kernel(table, pairs_flat)


def _tc_body(x_ref, ei_ref, op_ref, ee_ref,
             wg1, bg1, wg2, bg2,
             wl1a, wl1b, bl1, wl2, bl2,
             wc1a, wc1b, bc1, wc2, bc2,
             wm1a, wm1b, bm1, wm2, bm2,
             lr_ref, cr_ref, mr_ref):
    src = ei_ref[0, 0:1, :]
    dst = ei_ref[0, 1:2, :]
    sub_iota = lax.broadcasted_iota(jnp.int32, (_N, _E), 0)
    st = (src == sub_iota).astype(jnp.bfloat16)
    dt = (dst == sub_iota).astype(jnp.bfloat16)
    a = lax.dot_general(dt, st, (((1,), (1,)), ((), ())),
                        preferred_element_type=jnp.float32)
    deg = jnp.maximum(jnp.sum(a, axis=1, keepdims=True), 1.0)

    h = x_ref[0]
    m = jnp.dot(a, h, preferred_element_type=jnp.float32) / deg
    h = jnp.maximum(jnp.dot(m, wg1[...], preferred_element_type=jnp.float32)
                    + bg1[...], 0.0)
    m = jnp.dot(a, h, preferred_element_type=jnp.float32) / deg
    h = jnp.maximum(jnp.dot(m, wg2[...], preferred_element_type=jnp.float32)
                    + bg2[...], 0.0)

    p0 = op_ref[0][:, 0:1]
    p1 = op_ref[0][:, 1:2]
    lane_n = lax.broadcasted_iota(jnp.int32, (_P, _N), 1)
    ohp = (p0 == lane_n).astype(jnp.float32) + (p1 == lane_n).astype(jnp.float32)
    emb = jnp.dot(ohp, h, preferred_element_type=jnp.float32)
    ee = ee_ref[0]

    def head(wa, wb, b1, w2, b2):
        hh = jnp.maximum(
            jnp.dot(emb, wa[...], preferred_element_type=jnp.float32)
            + jnp.dot(ee, wb[...], preferred_element_type=jnp.float32)
            + b1[...], 0.0)
        return jnp.dot(hh, w2[...], preferred_element_type=jnp.float32) + b2[...]

    lr_ref[0] = head(wl1a, wl1b, bl1, wl2, bl2)
    cr_ref[0] = head(wc1a, wc1b, bc1, wc2, bc2)
    mr_ref[0] = head(wm1a, wm1b, bm1, wm2, bm2)


def _full(shape):
    return pl.BlockSpec(shape, lambda b: (0,) * len(shape))


def kernel(concatenated_node_features, edge_index, interaction_feature,
           object_pairs,
           W_g1, b_g1, W_g2, b_g2,
           W_cr1, b_cr1, W_cr2, b_cr2,
           W_lr1, b_lr1, W_lr2, b_lr2,
           W_mr1, b_mr1, W_mr2, b_mr2):
    x = concatenated_node_features

    table = interaction_feature.reshape(_B * _N * _N, _EF)
    ee = _edge_feature_gather(table, object_pairs.reshape(-1))
    ee = ee.reshape(_B, _P, _EF)

    heads = []
    for w1, b1, w2, b2 in ((W_lr1, b_lr1, W_lr2, b_lr2),
                           (W_cr1, b_cr1, W_cr2, b_cr2),
                           (W_mr1, b_mr1, W_mr2, b_mr2)):
        heads += [w1[:_D], w1[_D:], b1.reshape(1, -1), w2, b2.reshape(1, -1)]

    out_shapes = [jax.ShapeDtypeStruct((_B, _P, c), jnp.float32)
                  for c in (10, 26, 5)]
    per_b = lambda shape: pl.BlockSpec(shape, lambda b: (b, 0, 0))

    lr, cr, mr = pl.pallas_call(
        _tc_body,
        grid=(_B,),
        in_specs=[
            per_b((1, _N, _D)),
            per_b((1, 2, _E)),
            per_b((1, _P, 2)),
            per_b((1, _P, _EF)),
            _full((_D, _D)), _full((1, _D)), _full((_D, _D)), _full((1, _D)),
            _full((_D, 64)), _full((_EF, 64)), _full((1, 64)),
            _full((64, 10)), _full((1, 10)),
            _full((_D, 64)), _full((_EF, 64)), _full((1, 64)),
            _full((64, 26)), _full((1, 26)),
            _full((_D, 64)), _full((_EF, 64)), _full((1, 64)),
            _full((64, 5)), _full((1, 5)),
        ],
        out_specs=[per_b((1, _P, 10)), per_b((1, _P, 26)), per_b((1, _P, 5))],
        out_shape=out_shapes,
    )(x, edge_index, object_pairs, ee,
      W_g1, b_g1.reshape(1, -1), W_g2, b_g2.reshape(1, -1), *heads)
    return (lr, cr, mr)


# speedup vs baseline: 20.7621x; 2.8369x over previous
import functools

import jax
import jax.numpy as jnp
from jax import lax
from jax.experimental import pallas as pl
from jax.experimental.pallas import tpu as pltpu
from jax.experimental.pallas import tpu_sc as plsc

_B, _N, _E, _D, _EF, _P = 64, 128, 4096, 128, 16, 512
_NC, _NS = 2, 16
_NW = _NC * _NS
_PAIRS = _B * _P
_PW = _PAIRS // _NW
_IDXW = _PW * _EF
_RPD = 128
_NDMA = _IDXW // _RPD


def _edge_feature_gather(table_flat, pairs_lin):
    mesh = plsc.VectorSubcoreMesh(core_axis_name="c", subcore_axis_name="s")

    @functools.partial(
        pl.kernel,
        out_type=jax.ShapeDtypeStruct((_PAIRS * _EF,), jnp.float32),
        mesh=mesh,
        scratch_types=[
            pltpu.VMEM((2 * _PW,), jnp.int32),
            pltpu.VMEM((_NDMA, _RPD), jnp.int32),
            pltpu.VMEM((_IDXW,), jnp.float32),
            pltpu.SemaphoreType.DMA,
        ],
        compiler_params=pltpu.CompilerParams(use_tc_tiling_on_sc=False,
                                             needs_layout_passes=False),
    )
    def gather_kernel(table_hbm, pairs_hbm, out_hbm, pq_v, idx_v, rows_v, sem):
        wid = lax.axis_index("s") * _NC + lax.axis_index("c")
        pltpu.sync_copy(pairs_hbm.at[pl.ds(wid * 2 * _PW, 2 * _PW)], pq_v)
        lane = lax.iota(jnp.int32, 16)
        for c in range(_PW // 16):
            blo = (c // 32) * 1024 + ((c % 32) // 8) * 256 + (c % 8) * 16
            i0 = pq_v[pl.ds(blo, 16)]
            i1 = pq_v[pl.ds(blo + 128, 16)]
            b = wid * (_PW // _P) + c // 32
            a_c = (jnp.full((16,), b * (_N * _EF * _N), jnp.int32)
                   + i0 * (_EF * _N) + i1)

            def fbody(f, _):
                fv = jnp.full((16,), f, jnp.int32)
                pos = jnp.full((16,), c * 256, jnp.int32) + fv + lane * 16
                plsc.store_scatter(idx_v, [pos >> 7, pos & 127],
                                   a_c + fv * _N)
                return 0

            lax.fori_loop(0, _EF, fbody, 0)
        copies = [
            pltpu.async_copy(table_hbm.at[idx_v.at[k]],
                             rows_v.at[pl.ds(k * _RPD, _RPD)], sem)
            for k in range(_NDMA)
        ]
        for cp in copies:
            cp.wait()
        pltpu.sync_copy(rows_v, out_hbm.at[pl.ds(wid * _IDXW, _IDXW)])

    return gather_kernel(table_flat, pairs_lin)


def _tc_body(x_ref, ei_ref, op_ref, ee_ref,
             wg1, bg1, wg2, bg2,
             wl1a, wl1b, bl1, wl2, bl2,
             wc1a, wc1b, bc1, wc2, bc2,
             wm1a, wm1b, bm1, wm2, bm2,
             lr_ref, cr_ref, mr_ref):
    src = ei_ref[0, 0:1, :]
    dst = ei_ref[0, 1:2, :]
    sub_iota = lax.broadcasted_iota(jnp.int32, (_N, _E), 0)
    st = (src == sub_iota).astype(jnp.bfloat16)
    dt = (dst == sub_iota).astype(jnp.bfloat16)
    a = lax.dot_general(dt, st, (((1,), (1,)), ((), ())),
                        preferred_element_type=jnp.float32)
    deg = jnp.maximum(jnp.sum(a, axis=1, keepdims=True), 1.0)

    h = x_ref[0]
    m = jnp.dot(a, h, preferred_element_type=jnp.float32) / deg
    h = jnp.maximum(jnp.dot(m, wg1[...], preferred_element_type=jnp.float32)
                    + bg1[...], 0.0)
    m = jnp.dot(a, h, preferred_element_type=jnp.float32) / deg
    h = jnp.maximum(jnp.dot(m, wg2[...], preferred_element_type=jnp.float32)
                    + bg2[...], 0.0)

    p0 = op_ref[0][:, 0:1]
    p1 = op_ref[0][:, 1:2]
    lane_n = lax.broadcasted_iota(jnp.int32, (_P, _N), 1)
    ohp = (p0 == lane_n).astype(jnp.float32) + (p1 == lane_n).astype(jnp.float32)
    emb = jnp.dot(ohp, h, preferred_element_type=jnp.float32)
    ee = ee_ref[0]

    def head(wa, wb, b1, w2, b2, out_ref):
        hh = jnp.maximum(
            jnp.dot(emb, wa[...], preferred_element_type=jnp.float32)
            + jnp.dot(ee, wb[...], preferred_element_type=jnp.float32)
            + b1[...], 0.0)
        out_ref[:, 0, 0, :] = (
            lax.dot_general(w2[...], hh, (((0,), (1,)), ((), ())),
                            preferred_element_type=jnp.float32) + b2[...])

    head(wl1a, wl1b, bl1, wl2, bl2, lr_ref)
    head(wc1a, wc1b, bc1, wc2, bc2, cr_ref)
    head(wm1a, wm1b, bm1, wm2, bm2, mr_ref)


def _full(shape):
    return pl.BlockSpec(shape, lambda b: (0,) * len(shape))


def kernel(concatenated_node_features, edge_index, interaction_feature,
           object_pairs,
           W_g1, b_g1, W_g2, b_g2,
           W_cr1, b_cr1, W_cr2, b_cr2,
           W_lr1, b_lr1, W_lr2, b_lr2,
           W_mr1, b_mr1, W_mr2, b_mr2):
    x = concatenated_node_features

    table_flat = interaction_feature.transpose(0, 1, 3, 2).reshape(-1)
    pairs_lin = (object_pairs.reshape(_B, _P // _N, _N, 2)
                 .transpose(0, 1, 3, 2).reshape(-1))
    ee = _edge_feature_gather(table_flat, pairs_lin).reshape(_B, _P, _EF)

    heads = []
    for w1, b1, w2, b2 in ((W_lr1, b_lr1, W_lr2, b_lr2),
                           (W_cr1, b_cr1, W_cr2, b_cr2),
                           (W_mr1, b_mr1, W_mr2, b_mr2)):
        heads += [w1[:_D], w1[_D:], b1.reshape(1, -1), w2, b2.reshape(-1, 1)]

    out_shapes = [jax.ShapeDtypeStruct((c, _B, 1, _P), jnp.float32)
                  for c in (10, 26, 5)]
    per_b = lambda shape: pl.BlockSpec(shape, lambda b: (b, 0, 0))

    lrt, crt, mrt = pl.pallas_call(
        _tc_body,
        grid=(_B,),
        in_specs=[
            per_b((1, _N, _D)),
            per_b((1, 2, _E)),
            per_b((1, _P, 2)),
            per_b((1, _P, _EF)),
            _full((_D, _D)), _full((1, _D)), _full((_D, _D)), _full((1, _D)),
            _full((_D, 64)), _full((_EF, 64)), _full((1, 64)),
            _full((64, 10)), _full((10, 1)),
            _full((_D, 64)), _full((_EF, 64)), _full((1, 64)),
            _full((64, 26)), _full((26, 1)),
            _full((_D, 64)), _full((_EF, 64)), _full((1, 64)),
            _full((64, 5)), _full((5, 1)),
        ],
        out_specs=[pl.BlockSpec((10, 1, 1, _P), lambda b: (0, b, 0, 0)),
                   pl.BlockSpec((26, 1, 1, _P), lambda b: (0, b, 0, 0)),
                   pl.BlockSpec((5, 1, 1, _P), lambda b: (0, b, 0, 0))],
        out_shape=out_shapes,
    )(x, edge_index, object_pairs, ee,
      W_g1, b_g1.reshape(1, -1), W_g2, b_g2.reshape(1, -1), *heads)
    lr = lrt.reshape(10, _B, _P).transpose(1, 2, 0)
    cr = crt.reshape(26, _B, _P).transpose(1, 2, 0)
    mr = mrt.reshape(5, _B, _P).transpose(1, 2, 0)
    return (lr, cr, mr)


# speedup vs baseline: 24.8727x; 1.1980x over previous
import functools

import jax
import jax.numpy as jnp
from jax import lax
from jax.experimental import pallas as pl
from jax.experimental.pallas import tpu as pltpu
from jax.experimental.pallas import tpu_sc as plsc

_B, _N, _E, _D, _EF, _P = 64, 128, 4096, 128, 16, 512
_NC, _NS = 2, 16
_NW = _NC * _NS
_PAIRS = _B * _P
_PW = _PAIRS // _NW
_IDXW = _PW * _EF
_RPD = 128
_NDMA = _IDXW // _RPD


def _edge_feature_gather(table_flat, pairs_lin):
    mesh = plsc.VectorSubcoreMesh(core_axis_name="c", subcore_axis_name="s")

    @functools.partial(
        pl.kernel,
        out_type=jax.ShapeDtypeStruct((_PAIRS * _EF,), jnp.float32),
        mesh=mesh,
        scratch_types=[
            pltpu.VMEM((2 * _PW,), jnp.int32),
            pltpu.VMEM((_NDMA, _RPD), jnp.int32),
            pltpu.VMEM((_IDXW,), jnp.float32),
            pltpu.SemaphoreType.DMA,
        ],
        compiler_params=pltpu.CompilerParams(use_tc_tiling_on_sc=False,
                                             needs_layout_passes=False),
    )
    def gather_kernel(table_hbm, pairs_hbm, out_hbm, pq_v, idx_v, rows_v, sem):
        wid = lax.axis_index("s") * _NC + lax.axis_index("c")
        pltpu.sync_copy(pairs_hbm.at[pl.ds(wid * 2 * _PW, 2 * _PW)], pq_v)
        lane = lax.iota(jnp.int32, 16)
        for c in range(_PW // 16):
            blo = (c // 32) * 1024 + ((c % 32) // 8) * 256 + (c % 8) * 16
            i0 = pq_v[pl.ds(blo, 16)]
            i1 = pq_v[pl.ds(blo + 128, 16)]
            b = wid * (_PW // _P) + c // 32
            a_c = (jnp.full((16,), b * (_N * _EF * _N), jnp.int32)
                   + i0 * (_EF * _N) + i1)

            def fbody(f, _):
                fv = jnp.full((16,), f, jnp.int32)
                pos = jnp.full((16,), c * 256, jnp.int32) + fv + lane * 16
                plsc.store_scatter(idx_v, [pos >> 7, pos & 127],
                                   a_c + fv * _N)
                return 0

            lax.fori_loop(0, _EF, fbody, 0)
        copies = [
            pltpu.async_copy(table_hbm.at[idx_v.at[k]],
                             rows_v.at[pl.ds(k * _RPD, _RPD)], sem)
            for k in range(_NDMA)
        ]
        for cp in copies:
            cp.wait()
        pltpu.sync_copy(rows_v, out_hbm.at[pl.ds(wid * _IDXW, _IDXW)])

    return gather_kernel(table_flat, pairs_lin)


_BPS = 2


def _tc_body(x_ref, ei_ref, op_ref, ee_ref,
             wg1, bg1, wg2, bg2,
             wa_cat, wb_cat, b1_cat, w2_blk, b2_blk,
             lr_ref, cr_ref, mr_ref):
    for i in range(_BPS):
        src = ei_ref[i, 0:1, :]
        dst = ei_ref[i, 1:2, :]
        sub_iota = lax.broadcasted_iota(jnp.int32, (_N, _E), 0)
        st = (src == sub_iota).astype(jnp.bfloat16)
        dt = (dst == sub_iota).astype(jnp.bfloat16)
        a = lax.dot_general(dt, st, (((1,), (1,)), ((), ())),
                            preferred_element_type=jnp.float32)
        deg = jnp.maximum(
            jnp.dot(dt, jnp.ones((_E, 1), jnp.bfloat16),
                    preferred_element_type=jnp.float32), 1.0)

        h = x_ref[i]
        m = jnp.dot(a, h, preferred_element_type=jnp.float32) / deg
        h = jnp.maximum(
            jnp.dot(m, wg1[...], preferred_element_type=jnp.float32)
            + bg1[...], 0.0)
        m = jnp.dot(a, h, preferred_element_type=jnp.float32) / deg
        h = jnp.maximum(
            jnp.dot(m, wg2[...], preferred_element_type=jnp.float32)
            + bg2[...], 0.0)

        p0 = op_ref[i][:, 0:1]
        p1 = op_ref[i][:, 1:2]
        lane_n = lax.broadcasted_iota(jnp.int32, (_P, _N), 1)
        ohp = ((p0 == lane_n).astype(jnp.float32)
               + (p1 == lane_n).astype(jnp.float32))
        emb = jnp.dot(ohp, h, preferred_element_type=jnp.float32)
        ee = ee_ref[i]

        hh = jnp.maximum(
            jnp.dot(emb, wa_cat[...], preferred_element_type=jnp.float32)
            + jnp.dot(ee, wb_cat[...], preferred_element_type=jnp.float32)
            + b1_cat[...], 0.0)
        hall = (lax.dot_general(w2_blk[...], hh, (((0,), (1,)), ((), ())),
                                preferred_element_type=jnp.float32)
                + b2_blk[...])
        lr_ref[:, i, 0, :] = hall[0:10]
        cr_ref[:, i, 0, :] = hall[16:42]
        mr_ref[:, i, 0, :] = hall[48:53]


def _full(shape):
    return pl.BlockSpec(shape, lambda b: (0,) * len(shape))


def kernel(concatenated_node_features, edge_index, interaction_feature,
           object_pairs,
           W_g1, b_g1, W_g2, b_g2,
           W_cr1, b_cr1, W_cr2, b_cr2,
           W_lr1, b_lr1, W_lr2, b_lr2,
           W_mr1, b_mr1, W_mr2, b_mr2):
    x = concatenated_node_features

    table_flat = interaction_feature.transpose(0, 1, 3, 2).reshape(-1)
    pairs_lin = (object_pairs.reshape(_B, _P // _N, _N, 2)
                 .transpose(0, 1, 3, 2).reshape(-1))
    ee = _edge_feature_gather(table_flat, pairs_lin).reshape(_B, _P, _EF)

    wa_cat = jnp.concatenate([W_lr1[:_D], W_cr1[:_D], W_mr1[:_D]], axis=1)
    wb_cat = jnp.concatenate([W_lr1[_D:], W_cr1[_D:], W_mr1[_D:]], axis=1)
    b1_cat = jnp.concatenate([b_lr1, b_cr1, b_mr1]).reshape(1, -1)
    w2_blk = jnp.zeros((192, 56), jnp.float32)
    w2_blk = w2_blk.at[0:64, 0:10].set(W_lr2)
    w2_blk = w2_blk.at[64:128, 16:42].set(W_cr2)
    w2_blk = w2_blk.at[128:192, 48:53].set(W_mr2)
    b2_blk = jnp.zeros((56, 1), jnp.float32)
    b2_blk = b2_blk.at[0:10, 0].set(b_lr2)
    b2_blk = b2_blk.at[16:42, 0].set(b_cr2)
    b2_blk = b2_blk.at[48:53, 0].set(b_mr2)

    out_shapes = [jax.ShapeDtypeStruct((c, _B, 1, _P), jnp.float32)
                  for c in (10, 26, 5)]
    per_b = lambda shape: pl.BlockSpec(shape, lambda b: (b, 0, 0))

    lrt, crt, mrt = pl.pallas_call(
        _tc_body,
        grid=(_B // _BPS,),
        in_specs=[
            per_b((_BPS, _N, _D)),
            per_b((_BPS, 2, _E)),
            per_b((_BPS, _P, 2)),
            per_b((_BPS, _P, _EF)),
            _full((_D, _D)), _full((1, _D)), _full((_D, _D)), _full((1, _D)),
            _full((_D, 192)), _full((_EF, 192)), _full((1, 192)),
            _full((192, 56)), _full((56, 1)),
        ],
        out_specs=[pl.BlockSpec((10, _BPS, 1, _P), lambda b: (0, b, 0, 0)),
                   pl.BlockSpec((26, _BPS, 1, _P), lambda b: (0, b, 0, 0)),
                   pl.BlockSpec((5, _BPS, 1, _P), lambda b: (0, b, 0, 0))],
        out_shape=out_shapes,
    )(x, edge_index, object_pairs, ee,
      W_g1, b_g1.reshape(1, -1), W_g2, b_g2.reshape(1, -1),
      wa_cat, wb_cat, b1_cat, w2_blk, b2_blk)
    lr = lrt.reshape(10, _B, _P).transpose(1, 2, 0)
    cr = crt.reshape(26, _B, _P).transpose(1, 2, 0)
    mr = mrt.reshape(5, _B, _P).transpose(1, 2, 0)
    return (lr, cr, mr)


# speedup vs baseline: 27.4099x; 1.1020x over previous
import functools

import jax
import jax.numpy as jnp
from jax import lax
from jax.experimental import pallas as pl
from jax.experimental.pallas import tpu as pltpu
from jax.experimental.pallas import tpu_sc as plsc

_B, _N, _E, _D, _EF, _P = 64, 128, 4096, 128, 16, 512
_NC, _NS = 2, 16
_NW = _NC * _NS
_PAIRS = _B * _P
_PW = _PAIRS // _NW
_IDXW = _PW * _EF
_RPD = 128
_NDMA = _IDXW // _RPD


def _edge_feature_gather(table_flat, pairs_lin):
    mesh = plsc.VectorSubcoreMesh(core_axis_name="c", subcore_axis_name="s")

    @functools.partial(
        pl.kernel,
        out_type=jax.ShapeDtypeStruct((_EF, _PAIRS), jnp.float32),
        mesh=mesh,
        scratch_types=[
            pltpu.VMEM((2 * _PW,), jnp.int32),
            pltpu.VMEM((_NDMA, _RPD), jnp.int32),
            pltpu.VMEM((_IDXW,), jnp.float32),
            pltpu.SemaphoreType.DMA,
        ],
        compiler_params=pltpu.CompilerParams(use_tc_tiling_on_sc=False,
                                             needs_layout_passes=False),
    )
    def gather_kernel(table_hbm, pairs_hbm, out_hbm, pq_v, idx_v, rows_v, sem):
        wid = lax.axis_index("s") * _NC + lax.axis_index("c")
        pltpu.sync_copy(pairs_hbm.at[pl.ds(wid * 2 * _PW, 2 * _PW)], pq_v)
        copies = []
        for w in range(8):
            for c8 in range(8):
                c = w * 8 + c8
                blo = (c // 32) * 1024 + ((c % 32) // 8) * 256 + (c % 8) * 16
                i0 = pq_v[pl.ds(blo, 16)]
                i1 = pq_v[pl.ds(blo + 128, 16)]
                b = wid * (_PW // _P) + c // 32
                a_c = (jnp.full((16,), b * (_N * _EF * _N), jnp.int32)
                       + i0 * (_EF * _N) + i1)

                def fbody(f, _):
                    idx_v[f * 8 + w, pl.ds(c8 * 16, 16)] = (
                        a_c + jnp.full((16,), f, jnp.int32) * _N)
                    return 0

                lax.fori_loop(0, _EF, fbody, 0)
            for f in range(_EF):
                k = f * 8 + w
                copies.append(
                    pltpu.async_copy(table_hbm.at[idx_v.at[k]],
                                     rows_v.at[pl.ds(k * _RPD, _RPD)], sem))
        for cp in copies:
            cp.wait()
        for f in range(_EF):
            pltpu.sync_copy(rows_v.at[pl.ds(f * _PW, _PW)],
                            out_hbm.at[f, pl.ds(wid * _PW, _PW)])

    return gather_kernel(table_flat, pairs_lin)


_BPS = 4


def _tc_body(x_ref, ei_ref, op_ref, ee_ref,
             wg1, bg1, wg2, bg2,
             wa_cat, wb_cat, b1_cat, w2_blk, b2_blk,
             lr_ref, cr_ref, mr_ref):
    for i in range(_BPS):
        src = ei_ref[i, 0:1, :]
        dst = ei_ref[i, 1:2, :]
        sub_iota = lax.broadcasted_iota(jnp.int32, (_N, _E), 0)
        st = (src == sub_iota).astype(jnp.bfloat16)
        dt = (dst == sub_iota).astype(jnp.bfloat16)
        a = lax.dot_general(dt, st, (((1,), (1,)), ((), ())),
                            preferred_element_type=jnp.float32)
        deg = jnp.maximum(
            jnp.dot(dt, jnp.ones((_E, 1), jnp.bfloat16),
                    preferred_element_type=jnp.float32), 1.0)

        h = x_ref[i]
        m = jnp.dot(a, h, preferred_element_type=jnp.float32) / deg
        h = jnp.maximum(
            jnp.dot(m, wg1[...], preferred_element_type=jnp.float32)
            + bg1[...], 0.0)
        m = jnp.dot(a, h, preferred_element_type=jnp.float32) / deg
        h = jnp.maximum(
            jnp.dot(m, wg2[...], preferred_element_type=jnp.float32)
            + bg2[...], 0.0)

        p0 = op_ref[i][:, 0:1]
        p1 = op_ref[i][:, 1:2]
        lane_n = lax.broadcasted_iota(jnp.int32, (_P, _N), 1)
        ohp = ((p0 == lane_n).astype(jnp.float32)
               + (p1 == lane_n).astype(jnp.float32))
        emb = jnp.dot(ohp, h, preferred_element_type=jnp.float32)
        eet = ee_ref[:, i, 0, :]

        hh = jnp.maximum(
            jnp.dot(emb, wa_cat[...], preferred_element_type=jnp.float32)
            + lax.dot_general(eet, wb_cat[...], (((0,), (0,)), ((), ())),
                              preferred_element_type=jnp.float32)
            + b1_cat[...], 0.0)
        hall = (lax.dot_general(w2_blk[...], hh, (((0,), (1,)), ((), ())),
                                preferred_element_type=jnp.float32)
                + b2_blk[...])
        lr_ref[:, i, 0, :] = hall[0:10]
        cr_ref[:, i, 0, :] = hall[16:42]
        mr_ref[:, i, 0, :] = hall[48:53]


def _full(shape):
    return pl.BlockSpec(shape, lambda b: (0,) * len(shape))


def kernel(concatenated_node_features, edge_index, interaction_feature,
           object_pairs,
           W_g1, b_g1, W_g2, b_g2,
           W_cr1, b_cr1, W_cr2, b_cr2,
           W_lr1, b_lr1, W_lr2, b_lr2,
           W_mr1, b_mr1, W_mr2, b_mr2):
    x = concatenated_node_features

    table_flat = interaction_feature.transpose(0, 1, 3, 2).reshape(-1)
    pairs_lin = (object_pairs.reshape(_B, _P // _N, _N, 2)
                 .transpose(0, 1, 3, 2).reshape(-1))
    eet = _edge_feature_gather(table_flat, pairs_lin).reshape(_EF, _B, 1, _P)

    wa_cat = jnp.concatenate([W_lr1[:_D], W_cr1[:_D], W_mr1[:_D]], axis=1)
    wb_cat = jnp.concatenate([W_lr1[_D:], W_cr1[_D:], W_mr1[_D:]], axis=1)
    b1_cat = jnp.concatenate([b_lr1, b_cr1, b_mr1]).reshape(1, -1)
    w2_blk = jnp.zeros((192, 56), jnp.float32)
    w2_blk = w2_blk.at[0:64, 0:10].set(W_lr2)
    w2_blk = w2_blk.at[64:128, 16:42].set(W_cr2)
    w2_blk = w2_blk.at[128:192, 48:53].set(W_mr2)
    b2_blk = jnp.zeros((56, 1), jnp.float32)
    b2_blk = b2_blk.at[0:10, 0].set(b_lr2)
    b2_blk = b2_blk.at[16:42, 0].set(b_cr2)
    b2_blk = b2_blk.at[48:53, 0].set(b_mr2)

    out_shapes = [jax.ShapeDtypeStruct((c, _B, 1, _P), jnp.float32)
                  for c in (10, 26, 5)]
    per_b = lambda shape: pl.BlockSpec(shape, lambda b: (b, 0, 0))

    lrt, crt, mrt = pl.pallas_call(
        _tc_body,
        grid=(_B // _BPS,),
        in_specs=[
            per_b((_BPS, _N, _D)),
            per_b((_BPS, 2, _E)),
            per_b((_BPS, _P, 2)),
            pl.BlockSpec((_EF, _BPS, 1, _P), lambda b: (0, b, 0, 0)),
            _full((_D, _D)), _full((1, _D)), _full((_D, _D)), _full((1, _D)),
            _full((_D, 192)), _full((_EF, 192)), _full((1, 192)),
            _full((192, 56)), _full((56, 1)),
        ],
        out_specs=[pl.BlockSpec((10, _BPS, 1, _P), lambda b: (0, b, 0, 0)),
                   pl.BlockSpec((26, _BPS, 1, _P), lambda b: (0, b, 0, 0)),
                   pl.BlockSpec((5, _BPS, 1, _P), lambda b: (0, b, 0, 0))],
        out_shape=out_shapes,
    )(x, edge_index, object_pairs, eet,
      W_g1, b_g1.reshape(1, -1), W_g2, b_g2.reshape(1, -1),
      wa_cat, wb_cat, b1_cat, w2_blk, b2_blk)
    lr = lrt.reshape(10, _B, _P).transpose(1, 2, 0)
    cr = crt.reshape(26, _B, _P).transpose(1, 2, 0)
    mr = mrt.reshape(5, _B, _P).transpose(1, 2, 0)
    return (lr, cr, mr)
